# baseline (speedup 1.0000x reference)
import jax
import jax.numpy as jnp
from jax.experimental import pallas as pl


def _body(x_ref, t_ref, mask_ref, W1a_ref, W2T_ref, b2_ref, out_ref):
    _, bn, p = x_ref.shape
    b = bn * p
    x3 = x_ref[...]
    z2 = jnp.tanh(x3[0:2]).astype(jnp.bfloat16).reshape(2, b)
    t_bf = t_ref[...].astype(jnp.bfloat16)
    tT_bf = jnp.concatenate([t_bf[g] for g in range(bn)], axis=1)
    ones = jnp.ones((1, b), jnp.bfloat16)
    cat = jnp.concatenate([tT_bf, z2, ones], axis=0)
    W1a_bf = W1a_ref[...].astype(jnp.bfloat16)
    hT = jax.lax.dot_general(W1a_bf, cat, (((1,), (0,)), ((), ())),
                             preferred_element_type=jnp.float32)
    hT = jnp.maximum(hT, 0.0)
    pT = (W2T_ref[...] @ hT).reshape(10, bn, p) + b2_ref[...].reshape(10, 1, 1)
    dT = jax.nn.softplus(pT) + 1e-4
    xL1 = -dT[1:2]
    xL2 = xL1 - dT[0:1]
    xL3 = xL2 - 10000.0
    xR1 = dT[2:3]
    xR2 = xR1 + dT[3:4]
    xR3 = xR2 + 10000.0
    yL1 = -dT[5:6]
    yL2 = yL1 - dT[4:5]
    yL3 = yL2 - 20000.0 * dT[8:9]
    yR1 = dT[6:7]
    yR2 = yR1 + dT[7:8]
    yR3 = yR2 + 20000.0 * dT[9:10]
    ax = jnp.concatenate([xL3, xL2, xL1, xR1, xR2, xR3], axis=0)
    ay = jnp.concatenate([yL3, yL2, yL1, yR1, yR2, yR3], axis=0)
    qx = jnp.clip(x3[2:3], ax[0:1] * 0.99, ax[5:6] * 0.99)
    xl = ax[0:5]
    xr = ax[1:6]
    yl = ay[0:5]
    yr = ay[1:6]
    sel = ((qx >= xl) & (qx < xr)).astype(jnp.float32)
    xl_s = jnp.sum(xl * sel, axis=0, keepdims=True)
    xr_s = jnp.sum(xr * sel, axis=0, keepdims=True)
    yl_s = jnp.sum(yl * sel, axis=0, keepdims=True)
    yr_s = jnp.sum(yr * sel, axis=0, keepdims=True)
    gy = (yr_s - yl_s) / (xr_s - xl_s) * (qx - xl_s) + yl_s
    out_ref[...] = jnp.concatenate(
        [x3[0:2] * mask_ref[0:2].reshape(2, 1, 1), gy], axis=0)


def kernel(x, t_feat, mask, W1, b1, W2, b2):
    n, p, _ = x.shape
    rows = n * p
    blk = 32768
    bn = blk // p
    grid = rows // blk
    W1a = jnp.concatenate([W1[2:, :].T, W1[0:2, :].T, b1.reshape(128, 1)],
                          axis=1)
    x3 = jnp.transpose(x, (2, 0, 1))
    tt = jnp.transpose(t_feat, (0, 2, 1))
    const = lambda i: (0, 0)
    out = pl.pallas_call(
        _body,
        grid=(grid,),
        in_specs=[
            pl.BlockSpec((3, bn, p), lambda i: (0, i, 0)),
            pl.BlockSpec((bn, 64, p), lambda i: (i, 0, 0)),
            pl.BlockSpec((3, 1), const),
            pl.BlockSpec((128, 67), const),
            pl.BlockSpec((10, 128), const),
            pl.BlockSpec((10, 1), const),
        ],
        out_specs=pl.BlockSpec((3, bn, p), lambda i: (0, i, 0)),
        out_shape=jax.ShapeDtypeStruct((3, n, p), jnp.float32),
    )(x3, tt, mask.reshape(3, 1), W1a, W2.T, b2.reshape(10, 1))
    return jnp.transpose(out, (1, 2, 0))
